# baseline (speedup 1.0000x reference)
import functools

import jax
import jax.numpy as jnp
from jax import lax
from jax.experimental import pallas as pl
from jax.experimental.pallas import tpu as pltpu
from jax.experimental.pallas import tpu_sc as plsc

VOCAB = 1000000
EMB_DIM = 64
INPUT_LEN = 2048
LATENT_LEN = 512
OUTPUT_LEN = 256
BATCH = 4

NC = 2
NS = 16
L = 16
NW = NC * NS

B_TOK = BATCH * INPUT_LEN
CH = 256
N_FULL = VOCAB // CH
TAIL_C = N_FULL
TAIL_W = VOCAB - N_FULL * CH
X_ROWS = B_TOK + NW
X_COLS = 2 * EMB_DIM


def _sc_body(idx_hbm, table_hbm, tail_hbm, enc_hbm, proc_hbm, dec_hbm,
             x_out, lat_out, dec_out,
             idx_v, mtok, mpos, ctok, cpos, buf3, tbuf, flags,
             stage, ebuf, posbuf, pmod, gsem, esem):
    cid = lax.axis_index("c")
    sid = lax.axis_index("s")
    wid = sid * NC + cid
    iota = lax.iota(jnp.int32, L)

    pltpu.sync_copy(idx_hbm, idx_v)

    @pl.when(wid < 16)
    def _lat():
        b = wid // 4
        ch = wid % 4
        pltpu.sync_copy(proc_hbm.at[:, pl.ds(ch * 128, 128)], tbuf)
        pltpu.sync_copy(tbuf, lat_out.at[b, :, pl.ds(ch * 128, 128)])

    @pl.when((wid >= 16) & (wid < 24))
    def _dec():
        b = (wid - 16) // 2
        ch = (wid - 16) % 2
        pltpu.sync_copy(dec_hbm.at[:, pl.ds(ch * 128, 128)], tbuf)

        def dbl(r, _):
            for c4 in range(128 // L):
                sl = pl.ds(c4 * L, L)
                v = tbuf[r, sl]
                tbuf[r, sl] = v + v
            return 0

        lax.fori_loop(0, EMB_DIM, dbl, 0)
        pltpu.sync_copy(tbuf, dec_out.at[b, :, pl.ds(ch * 128, 128)])

    zero = jnp.zeros((L,), jnp.int32)

    def zf(g, _):
        flags[pl.ds(g * L, L)] = zero
        return 0

    lax.fori_loop(0, (VOCAB // 128 + 2 * L) // L, zf, 0)

    one = jnp.full((L,), 1, jnp.int32)

    def scan(g, off):
        v = idx_v[g // 8, pl.ds((g % 8) * L, L)]
        pv = g * L + iota
        m = ((v >> 8) & 31) == wid
        plsc.store_compressed(mtok.at[pl.ds(off, L)], v, mask=m)
        plsc.store_compressed(mpos.at[pl.ds(off, L)], pv, mask=m)
        plsc.store_scatter(flags, [v >> 7], one, mask=m)
        return off + plsc.all_reduce_population_count(m)[0]

    mcnt = lax.fori_loop(0, B_TOK // L, scan, 0)

    dump_row = B_TOK + wid
    nvreg = (mcnt + L - 1) >> 4

    def flush(pvs):
        for si in range(4):
            posbuf[0, pl.ds(si * L, L)] = pvs[si]
            pmod[0, pl.ds(si * L, L)] = pvs[si] & (INPUT_LEN - 1)
        pltpu.async_copy(enc_hbm.at[pmod.at[0]], ebuf, esem).wait()

        def addrow(r, _):
            for k4 in range(EMB_DIM // L):
                sl = pl.ds(k4 * L, L)
                stage[r, sl] = stage[r, sl] + ebuf[r, sl]
            return 0

        lax.fori_loop(0, 4 * L, addrow, 0)
        pltpu.sync_copy(stage, x_out.at[posbuf.at[0]])

    n_my_chunks = ((TAIL_C - wid) >> 5) + 1

    def start_dma(c, par):
        for q in range(2):
            f = flags[pl.ds(c * 2 + q, L)][0]

            @pl.when((f > 0) & (c != TAIL_C))
            def _s1():
                pltpu.async_copy(
                    table_hbm.at[:, pl.ds(c * CH + q * 128, 128)],
                    buf3.at[par, :, pl.ds(q * 128, 128)], gsem)

            @pl.when((f > 0) & (c == TAIL_C) & (q == 0))
            def _s2():
                pltpu.async_copy(tail_hbm, buf3.at[par, :, pl.ds(0, 128)],
                                 gsem)

    def wait_dma(c, par):
        for q in range(2):
            f = flags[pl.ds(c * 2 + q, L)][0]

            @pl.when(f > 0)
            def _w1():
                pltpu.make_async_copy(
                    table_hbm.at[:, pl.ds(0, 128)],
                    buf3.at[par, :, pl.ds(q * 128, 128)], gsem).wait()

    start_dma(wid, 0)
    start_dma(wid + NW, 1)
    start_dma(wid + NW * 2, 2)

    def chunk_step(i, carry):
        j16, posv = carry
        c = wid + NW * i
        cbase = c * CH
        par = i & 3

        @pl.when(i + 3 < n_my_chunks)
        def _pref():
            start_dma(wid + NW * (i + 3), (i + 3) & 3)

        wait_dma(c, par)
        parv = jnp.full((L,), par, jnp.int32)

        def list_step(g, carry2):
            jj, pv2 = carry2
            mt = mtok[pl.ds(g * L, L)]
            mp = mpos[pl.ds(g * L, L)]
            valid = (g * L + iota) < mcnt
            m = valid & ((mt >> 8) == c)
            plsc.store_compressed(ctok.at[pl.ds(0, L)], mt, mask=m)
            plsc.store_compressed(cpos.at[pl.ds(0, L)], mp, mask=m)
            k = plsc.all_reduce_population_count(m)[0]

            def entry(kk, carry3):
                j3, pvs = carry3
                tok = ctok[pl.ds(kk, L)][0]
                p = cpos[pl.ds(kk, L)][0]
                col = tok - cbase
                jm = j3 & (4 * L - 1)
                s = jm >> 4
                lane = jm & (L - 1)
                cvec = jnp.full((L,), col, jnp.int32)
                for k4 in range(EMB_DIM // L):
                    dvec = iota + k4 * L
                    vals = plsc.load_gather(buf3, [parv, dvec, cvec])
                    stage[jm, pl.ds(k4 * L, L)] = vals
                pvs = tuple(
                    jnp.where((iota == lane) & (s == si), p, pvs[si])
                    for si in range(4))

                @pl.when(jm == 4 * L - 1)
                def _f():
                    flush(pvs)

                dumpv = jnp.full((L,), dump_row, jnp.int32)
                pvs = tuple(
                    jnp.where(jm == 4 * L - 1, dumpv, pvs[si])
                    for si in range(4))
                return j3 + 1, pvs

            return lax.fori_loop(0, k, entry, (jj, pv2))

        return lax.fori_loop(0, nvreg, list_step, (j16, posv))

    posv0 = jnp.full((L,), dump_row, jnp.int32)
    j16, posvs = lax.fori_loop(0, n_my_chunks, chunk_step,
                               (0, (posv0, posv0, posv0, posv0)))

    flush(posvs)


@jax.jit
def _embed(idx_2d, table_t, tail_p, enc_p, proc_t, dec_t):
    mesh = plsc.VectorSubcoreMesh(core_axis_name="c", subcore_axis_name="s")
    out_type = [
        jax.ShapeDtypeStruct((X_ROWS, X_COLS), jnp.float32),
        jax.ShapeDtypeStruct((BATCH, EMB_DIM, LATENT_LEN), jnp.float32),
        jax.ShapeDtypeStruct((BATCH, EMB_DIM, OUTPUT_LEN), jnp.float32),
    ]
    scratch = [
        pltpu.VMEM((B_TOK // 128, 128), jnp.int32),
        pltpu.VMEM((B_TOK + L,), jnp.int32),
        pltpu.VMEM((B_TOK + L,), jnp.int32),
        pltpu.VMEM((2 * L,), jnp.int32),
        pltpu.VMEM((2 * L,), jnp.int32),
        pltpu.VMEM((4, EMB_DIM, CH), jnp.float32),
        pltpu.VMEM((EMB_DIM, 128), jnp.float32),
        pltpu.VMEM((VOCAB // 128 + 2 * L,), jnp.int32),
        pltpu.VMEM((4 * L, X_COLS), jnp.float32),
        pltpu.VMEM((4 * L, X_COLS), jnp.float32),
        pltpu.VMEM((1, 4 * L), jnp.int32),
        pltpu.VMEM((1, 4 * L), jnp.int32),
        pltpu.SemaphoreType.DMA,
        pltpu.SemaphoreType.DMA,
    ]
    run = functools.partial(
        pl.kernel, mesh=mesh, out_type=out_type, scratch_types=scratch,
        compiler_params=pltpu.CompilerParams(use_tc_tiling_on_sc=True, needs_layout_passes=False),
    )(_sc_body)
    return run(idx_2d, table_t, tail_p, enc_p, proc_t, dec_t)


def kernel(input_array, emb_table, pos_emb_encoder, pos_emb_processor, pos_emb_decoder):
    idx_2d = input_array.reshape(B_TOK // 128, 128)
    tail_p = jnp.pad(emb_table[N_FULL * CH:].T, ((0, 0), (0, X_COLS - TAIL_W)))
    enc_p = jnp.pad(pos_emb_encoder, ((0, 0), (0, X_COLS - EMB_DIM)))
    xs, lat_t, dec_t = _embed(idx_2d, emb_table.T, tail_p, enc_p,
                              pos_emb_processor.T, pos_emb_decoder.T)
    x = xs[:B_TOK, :EMB_DIM].reshape(BATCH, INPUT_LEN, EMB_DIM)
    return (
        x,
        lat_t.transpose(0, 2, 1),
        dec_t.transpose(0, 2, 1),
    )


# speedup vs baseline: 1.0080x; 1.0080x over previous
import functools

import jax
import jax.numpy as jnp
from jax import lax
from jax.experimental import pallas as pl
from jax.experimental.pallas import tpu as pltpu
from jax.experimental.pallas import tpu_sc as plsc

VOCAB = 1000000
EMB_DIM = 64
INPUT_LEN = 2048
LATENT_LEN = 512
OUTPUT_LEN = 256
BATCH = 4

NC = 2
NS = 16
L = 16
NW = NC * NS

B_TOK = BATCH * INPUT_LEN
CH = 512
N_FULL = VOCAB // CH
TAIL_C = N_FULL
TAIL_W = VOCAB - N_FULL * CH
X_ROWS = B_TOK + NW
X_COLS = 2 * EMB_DIM


def _sc_body(idx_hbm, table_hbm, tail_hbm, enc_hbm, proc_hbm, dec_hbm,
             x_out, lat_out, dec_out,
             idx_v, mtok, mpos, ctok, cpos, buf3, tbuf, flags,
             stage, ebuf, posbuf, pmod, gsem, esem):
    cid = lax.axis_index("c")
    sid = lax.axis_index("s")
    wid = sid * NC + cid
    iota = lax.iota(jnp.int32, L)

    pltpu.sync_copy(idx_hbm, idx_v)

    @pl.when(wid < 16)
    def _lat():
        b = wid // 4
        ch = wid % 4
        pltpu.sync_copy(proc_hbm.at[:, pl.ds(ch * 128, 128)], tbuf)
        pltpu.sync_copy(tbuf, lat_out.at[b, :, pl.ds(ch * 128, 128)])

    @pl.when((wid >= 16) & (wid < 24))
    def _dec():
        b = (wid - 16) // 2
        ch = (wid - 16) % 2
        pltpu.sync_copy(dec_hbm.at[:, pl.ds(ch * 128, 128)], tbuf)

        def dbl(r, _):
            for c4 in range(128 // L):
                sl = pl.ds(c4 * L, L)
                v = tbuf[r, sl]
                tbuf[r, sl] = v + v
            return 0

        lax.fori_loop(0, EMB_DIM, dbl, 0)
        pltpu.sync_copy(tbuf, dec_out.at[b, :, pl.ds(ch * 128, 128)])

    zero = jnp.zeros((L,), jnp.int32)

    def zf(g, _):
        flags[pl.ds(g * L, L)] = zero
        return 0

    lax.fori_loop(0, (VOCAB // 128 + 2 * L) // L, zf, 0)

    one = jnp.full((L,), 1, jnp.int32)

    def scan(g, off):
        v = idx_v[g // 8, pl.ds((g % 8) * L, L)]
        pv = g * L + iota
        m = ((v >> 9) & 31) == wid
        plsc.store_compressed(mtok.at[pl.ds(off, L)], v, mask=m)
        plsc.store_compressed(mpos.at[pl.ds(off, L)], pv, mask=m)
        plsc.store_scatter(flags, [v >> 7], one, mask=m)
        return off + plsc.all_reduce_population_count(m)[0]

    mcnt = lax.fori_loop(0, B_TOK // L, scan, 0)

    dump_row = B_TOK + wid
    nvreg = (mcnt + L - 1) >> 4

    def flush(pvs):
        for si in range(4):
            posbuf[0, pl.ds(si * L, L)] = pvs[si]
            pmod[0, pl.ds(si * L, L)] = pvs[si] & (INPUT_LEN - 1)
        pltpu.async_copy(enc_hbm.at[pmod.at[0]], ebuf, esem).wait()

        def addrow(r, _):
            for k4 in range(EMB_DIM // L):
                sl = pl.ds(k4 * L, L)
                stage[r, sl] = stage[r, sl] + ebuf[r, sl]
            return 0

        lax.fori_loop(0, 4 * L, addrow, 0)
        pltpu.sync_copy(stage, x_out.at[posbuf.at[0]])

    n_my_chunks = ((TAIL_C - wid) >> 5) + 1

    def start_dma(c, par):
        for q in range(4):
            f = flags[pl.ds(c * 4 + q, L)][0]

            @pl.when((f > 0) & (c != TAIL_C))
            def _s1():
                pltpu.async_copy(
                    table_hbm.at[:, pl.ds(c * CH + q * 128, 128)],
                    buf3.at[par, :, pl.ds(q * 128, 128)], gsem)

            @pl.when((f > 0) & (c == TAIL_C) & (q == 0))
            def _s2():
                pltpu.async_copy(tail_hbm, buf3.at[par, :, pl.ds(0, 128)],
                                 gsem)

    def wait_dma(c, par):
        for q in range(4):
            f = flags[pl.ds(c * 4 + q, L)][0]

            @pl.when(f > 0)
            def _w1():
                pltpu.make_async_copy(
                    table_hbm.at[:, pl.ds(0, 128)],
                    buf3.at[par, :, pl.ds(q * 128, 128)], gsem).wait()

    start_dma(wid, 0)

    def chunk_step(i, carry):
        j16, posv = carry
        c = wid + NW * i
        cbase = c * CH
        par = i & 1

        @pl.when(i + 1 < n_my_chunks)
        def _pref():
            start_dma(wid + NW * (i + 1), 1 - par)

        wait_dma(c, par)
        parv = jnp.full((L,), par, jnp.int32)

        def list_step(g, carry2):
            jj, pv2 = carry2
            mt = mtok[pl.ds(g * L, L)]
            mp = mpos[pl.ds(g * L, L)]
            valid = (g * L + iota) < mcnt
            m = valid & ((mt >> 9) == c)
            plsc.store_compressed(ctok.at[pl.ds(0, L)], mt, mask=m)
            plsc.store_compressed(cpos.at[pl.ds(0, L)], mp, mask=m)
            k = plsc.all_reduce_population_count(m)[0]

            def entry(kk, carry3):
                j3, pvs = carry3
                tok = ctok[pl.ds(kk, L)][0]
                p = cpos[pl.ds(kk, L)][0]
                col = tok - cbase
                jm = j3 & (4 * L - 1)
                s = jm >> 4
                lane = jm & (L - 1)
                cvec = jnp.full((L,), col, jnp.int32)
                for k4 in range(EMB_DIM // L):
                    dvec = iota + k4 * L
                    vals = plsc.load_gather(buf3, [parv, dvec, cvec])
                    stage[jm, pl.ds(k4 * L, L)] = vals
                pvs = tuple(
                    jnp.where((iota == lane) & (s == si), p, pvs[si])
                    for si in range(4))

                @pl.when(jm == 4 * L - 1)
                def _f():
                    flush(pvs)

                dumpv = jnp.full((L,), dump_row, jnp.int32)
                pvs = tuple(
                    jnp.where(jm == 4 * L - 1, dumpv, pvs[si])
                    for si in range(4))
                return j3 + 1, pvs

            return lax.fori_loop(0, k, entry, (jj, pv2))

        return lax.fori_loop(0, nvreg, list_step, (j16, posv))

    posv0 = jnp.full((L,), dump_row, jnp.int32)
    j16, posvs = lax.fori_loop(0, n_my_chunks, chunk_step,
                               (0, (posv0, posv0, posv0, posv0)))

    flush(posvs)


@jax.jit
def _embed(idx_2d, table_t, tail_p, enc_p, proc_t, dec_t):
    mesh = plsc.VectorSubcoreMesh(core_axis_name="c", subcore_axis_name="s")
    out_type = [
        jax.ShapeDtypeStruct((X_ROWS, X_COLS), jnp.float32),
        jax.ShapeDtypeStruct((BATCH, EMB_DIM, LATENT_LEN), jnp.float32),
        jax.ShapeDtypeStruct((BATCH, EMB_DIM, OUTPUT_LEN), jnp.float32),
    ]
    scratch = [
        pltpu.VMEM((B_TOK // 128, 128), jnp.int32),
        pltpu.VMEM((B_TOK + L,), jnp.int32),
        pltpu.VMEM((B_TOK + L,), jnp.int32),
        pltpu.VMEM((2 * L,), jnp.int32),
        pltpu.VMEM((2 * L,), jnp.int32),
        pltpu.VMEM((2, EMB_DIM, CH), jnp.float32),
        pltpu.VMEM((EMB_DIM, 128), jnp.float32),
        pltpu.VMEM((VOCAB // 128 + 2 * L,), jnp.int32),
        pltpu.VMEM((4 * L, X_COLS), jnp.float32),
        pltpu.VMEM((4 * L, X_COLS), jnp.float32),
        pltpu.VMEM((1, 4 * L), jnp.int32),
        pltpu.VMEM((1, 4 * L), jnp.int32),
        pltpu.SemaphoreType.DMA,
        pltpu.SemaphoreType.DMA,
    ]
    run = functools.partial(
        pl.kernel, mesh=mesh, out_type=out_type, scratch_types=scratch,
        compiler_params=pltpu.CompilerParams(use_tc_tiling_on_sc=True, needs_layout_passes=False),
    )(_sc_body)
    return run(idx_2d, table_t, tail_p, enc_p, proc_t, dec_t)


def kernel(input_array, emb_table, pos_emb_encoder, pos_emb_processor, pos_emb_decoder):
    idx_2d = input_array.reshape(B_TOK // 128, 128)
    tail_p = jnp.pad(emb_table[N_FULL * CH:].T, ((0, 0), (0, X_COLS - TAIL_W)))
    enc_p = jnp.pad(pos_emb_encoder, ((0, 0), (0, X_COLS - EMB_DIM)))
    xs, lat_t, dec_t = _embed(idx_2d, emb_table.T, tail_p, enc_p,
                              pos_emb_processor.T, pos_emb_decoder.T)
    x = xs[:B_TOK, :EMB_DIM].reshape(BATCH, INPUT_LEN, EMB_DIM)
    return (
        x,
        lat_t.transpose(0, 2, 1),
        dec_t.transpose(0, 2, 1),
    )


# speedup vs baseline: 1.0160x; 1.0079x over previous
import functools

import jax
import jax.numpy as jnp
from jax import lax
from jax.experimental import pallas as pl
from jax.experimental.pallas import tpu as pltpu
from jax.experimental.pallas import tpu_sc as plsc

VOCAB = 1000000
EMB_DIM = 64
INPUT_LEN = 2048
LATENT_LEN = 512
OUTPUT_LEN = 256
BATCH = 4

NC = 2
NS = 16
L = 16
NW = NC * NS

B_TOK = BATCH * INPUT_LEN
CH = 512
N_FULL = VOCAB // CH
TAIL_C = N_FULL
TAIL_W = VOCAB - N_FULL * CH
X_ROWS = B_TOK + NW
X_COLS = 2 * EMB_DIM


def _sc_body(idx_hbm, table_hbm, tail_hbm, enc_hbm, proc_hbm, dec_hbm,
             x_out, lat_out, dec_out,
             idx_v, mtok, mpos, ctok, cpos, buf3, tbuf, flags,
             stage, ebuf, posbuf, pmod, gsem, esem):
    cid = lax.axis_index("c")
    sid = lax.axis_index("s")
    wid = sid * NC + cid
    iota = lax.iota(jnp.int32, L)

    pltpu.async_copy(table_hbm.at[:, pl.ds(wid * CH, CH)], buf3.at[0], gsem)

    pltpu.sync_copy(idx_hbm, idx_v)

    @pl.when(wid < 16)
    def _lat():
        b = wid // 4
        ch = wid % 4
        pltpu.sync_copy(proc_hbm.at[:, pl.ds(ch * 128, 128)], tbuf)
        pltpu.sync_copy(tbuf, lat_out.at[b, :, pl.ds(ch * 128, 128)])

    @pl.when((wid >= 16) & (wid < 24))
    def _dec():
        b = (wid - 16) // 2
        ch = (wid - 16) % 2
        pltpu.sync_copy(dec_hbm.at[:, pl.ds(ch * 128, 128)], tbuf)

        def dbl(r, _):
            for c4 in range(128 // L):
                sl = pl.ds(c4 * L, L)
                v = tbuf[r, sl]
                tbuf[r, sl] = v + v
            return 0

        lax.fori_loop(0, EMB_DIM, dbl, 0)
        pltpu.sync_copy(tbuf, dec_out.at[b, :, pl.ds(ch * 128, 128)])

    zero = jnp.zeros((L,), jnp.int32)

    def zf(g, _):
        flags[pl.ds(g * L, L)] = zero
        return 0

    lax.fori_loop(0, (VOCAB // 128 + 2 * L) // L, zf, 0)

    one = jnp.full((L,), 1, jnp.int32)

    def scan(g, off):
        v = idx_v[g // 8, pl.ds((g % 8) * L, L)]
        pv = g * L + iota
        m = ((v >> 9) & 31) == wid
        plsc.store_compressed(mtok.at[pl.ds(off, L)], v, mask=m)
        plsc.store_compressed(mpos.at[pl.ds(off, L)], pv, mask=m)
        plsc.store_scatter(flags, [v >> 7], one, mask=m)
        return off + plsc.all_reduce_population_count(m)[0]

    mcnt = lax.fori_loop(0, B_TOK // L, scan, 0)

    dump_row = B_TOK + wid
    nvreg = (mcnt + L - 1) >> 4

    def flush(pvs):
        for si in range(4):
            posbuf[0, pl.ds(si * L, L)] = pvs[si]
            pmod[0, pl.ds(si * L, L)] = pvs[si] & (INPUT_LEN - 1)
        pltpu.async_copy(enc_hbm.at[pmod.at[0]], ebuf, esem).wait()

        def addrow(r, _):
            for k4 in range(EMB_DIM // L):
                sl = pl.ds(k4 * L, L)
                stage[r, sl] = stage[r, sl] + ebuf[r, sl]
            return 0

        lax.fori_loop(0, 4 * L, addrow, 0)
        pltpu.sync_copy(stage, x_out.at[posbuf.at[0]])

    n_my_chunks = ((TAIL_C - wid) >> 5) + 1

    def start_dma(c, par):
        for q in range(4):
            f = flags[pl.ds(c * 4 + q, L)][0]

            @pl.when((f > 0) & (c != TAIL_C))
            def _s1():
                pltpu.async_copy(
                    table_hbm.at[:, pl.ds(c * CH + q * 128, 128)],
                    buf3.at[par, :, pl.ds(q * 128, 128)], gsem)

            @pl.when((f > 0) & (c == TAIL_C) & (q == 0))
            def _s2():
                pltpu.async_copy(tail_hbm, buf3.at[par, :, pl.ds(0, 128)],
                                 gsem)

    def wait_dma(c, par):
        for q in range(4):
            f = flags[pl.ds(c * 4 + q, L)][0]

            @pl.when(f > 0)
            def _w1():
                pltpu.make_async_copy(
                    table_hbm.at[:, pl.ds(0, 128)],
                    buf3.at[par, :, pl.ds(q * 128, 128)], gsem).wait()

    def chunk_step(i, carry):
        j16, posv = carry
        c = wid + NW * i
        cbase = c * CH
        par = i & 1

        @pl.when(i + 1 < n_my_chunks)
        def _pref():
            start_dma(wid + NW * (i + 1), 1 - par)

        @pl.when(i == 0)
        def _w0():
            pltpu.make_async_copy(table_hbm.at[:, pl.ds(0, CH)],
                                  buf3.at[par], gsem).wait()

        @pl.when(i != 0)
        def _wn():
            wait_dma(c, par)
        parv = jnp.full((L,), par, jnp.int32)

        def list_step(g, carry2):
            jj, pv2 = carry2
            mt = mtok[pl.ds(g * L, L)]
            mp = mpos[pl.ds(g * L, L)]
            valid = (g * L + iota) < mcnt
            m = valid & ((mt >> 9) == c)
            plsc.store_compressed(ctok.at[pl.ds(0, L)], mt, mask=m)
            plsc.store_compressed(cpos.at[pl.ds(0, L)], mp, mask=m)
            k = plsc.all_reduce_population_count(m)[0]

            def entry(kk, carry3):
                j3, pvs = carry3
                tok = ctok[pl.ds(kk, L)][0]
                p = cpos[pl.ds(kk, L)][0]
                col = tok - cbase
                jm = j3 & (4 * L - 1)
                s = jm >> 4
                lane = jm & (L - 1)
                cvec = jnp.full((L,), col, jnp.int32)
                for k4 in range(EMB_DIM // L):
                    dvec = iota + k4 * L
                    vals = plsc.load_gather(buf3, [parv, dvec, cvec])
                    stage[jm, pl.ds(k4 * L, L)] = vals
                pvs = tuple(
                    jnp.where((iota == lane) & (s == si), p, pvs[si])
                    for si in range(4))

                @pl.when(jm == 4 * L - 1)
                def _f():
                    flush(pvs)

                dumpv = jnp.full((L,), dump_row, jnp.int32)
                pvs = tuple(
                    jnp.where(jm == 4 * L - 1, dumpv, pvs[si])
                    for si in range(4))
                return j3 + 1, pvs

            return lax.fori_loop(0, k, entry, (jj, pv2))

        return lax.fori_loop(0, nvreg, list_step, (j16, posv))

    posv0 = jnp.full((L,), dump_row, jnp.int32)
    j16, posvs = lax.fori_loop(0, n_my_chunks, chunk_step,
                               (0, (posv0, posv0, posv0, posv0)))

    flush(posvs)


@jax.jit
def _embed(idx_2d, table_t, tail_p, enc_p, proc_t, dec_t):
    mesh = plsc.VectorSubcoreMesh(core_axis_name="c", subcore_axis_name="s")
    out_type = [
        jax.ShapeDtypeStruct((X_ROWS, X_COLS), jnp.float32),
        jax.ShapeDtypeStruct((BATCH, EMB_DIM, LATENT_LEN), jnp.float32),
        jax.ShapeDtypeStruct((BATCH, EMB_DIM, OUTPUT_LEN), jnp.float32),
    ]
    scratch = [
        pltpu.VMEM((B_TOK // 128, 128), jnp.int32),
        pltpu.VMEM((B_TOK + L,), jnp.int32),
        pltpu.VMEM((B_TOK + L,), jnp.int32),
        pltpu.VMEM((2 * L,), jnp.int32),
        pltpu.VMEM((2 * L,), jnp.int32),
        pltpu.VMEM((2, EMB_DIM, CH), jnp.float32),
        pltpu.VMEM((EMB_DIM, 128), jnp.float32),
        pltpu.VMEM((VOCAB // 128 + 2 * L,), jnp.int32),
        pltpu.VMEM((4 * L, X_COLS), jnp.float32),
        pltpu.VMEM((4 * L, X_COLS), jnp.float32),
        pltpu.VMEM((1, 4 * L), jnp.int32),
        pltpu.VMEM((1, 4 * L), jnp.int32),
        pltpu.SemaphoreType.DMA,
        pltpu.SemaphoreType.DMA,
    ]
    run = functools.partial(
        pl.kernel, mesh=mesh, out_type=out_type, scratch_types=scratch,
        compiler_params=pltpu.CompilerParams(use_tc_tiling_on_sc=True, needs_layout_passes=False),
    )(_sc_body)
    return run(idx_2d, table_t, tail_p, enc_p, proc_t, dec_t)


def kernel(input_array, emb_table, pos_emb_encoder, pos_emb_processor, pos_emb_decoder):
    idx_2d = input_array.reshape(B_TOK // 128, 128)
    tail_p = jnp.pad(emb_table[N_FULL * CH:].T, ((0, 0), (0, X_COLS - TAIL_W)))
    enc_p = jnp.pad(pos_emb_encoder, ((0, 0), (0, X_COLS - EMB_DIM)))
    xs, lat_t, dec_t = _embed(idx_2d, emb_table.T, tail_p, enc_p,
                              pos_emb_processor.T, pos_emb_decoder.T)
    x = xs[:B_TOK, :EMB_DIM].reshape(BATCH, INPUT_LEN, EMB_DIM)
    return (
        x,
        lat_t.transpose(0, 2, 1),
        dec_t.transpose(0, 2, 1),
    )
